# baseline (speedup 1.0000x reference)
import functools

import jax
import jax.numpy as jnp
from jax import lax
from jax.experimental import pallas as pl
from jax.experimental.pallas import tpu as pltpu
from jax.experimental.pallas import tpu_sc as plsc

NODE_NUM = 512
DIM = 128
INPUT_DIM = 5
TOPK = 30
BATCH = 32
INTER = 256
EPS = 1e-5
NEG_INF = float("-inf")

SC_CORES = 2
SC_SUBCORES = 16
NUM_WORKERS = SC_CORES * SC_SUBCORES
ROWS_PER_W = NODE_NUM // NUM_WORKERS
LANES = 16


def _cos_kernel(emb_ref, cos_ref):
    w = emb_ref[...]
    g = jnp.dot(w, w.T, preferred_element_type=jnp.float32)
    nrm = jnp.sqrt(jnp.sum(w * w, axis=1, keepdims=True))
    cos_ref[...] = g / (nrm * nrm.T)


_UNROLL = 8


def _sc_topk_body(cos_hbm, mask_hbm, t_v, k_v, hist_v, coarse_v, out_v):
    wid = lax.axis_index("s") * SC_CORES + lax.axis_index("c")
    base = wid * ROWS_PER_W
    pltpu.sync_copy(cos_hbm.at[pl.ds(base, ROWS_PER_W), :], t_v)

    lane = lax.iota(jnp.int32, LANES)
    ones = jnp.full((LANES,), 1, jnp.int32)
    zeros = jnp.full((LANES,), 0, jnp.int32)
    kvec = jnp.full((LANES,), TOPK, jnp.int32)

    def splat(v):
        return jnp.full((LANES,), v, jnp.int32)

    def zero_hists(i, _):
        for u in range(_UNROLL):
            plsc.store_scatter(hist_v, [splat(i), splat(u * LANES) + lane],
                               zeros)
        return 0

    def zero_coarse(i, _):
        for u in range(_UNROLL):
            plsc.store_scatter(coarse_v, [splat(i), splat(u * LANES) + lane],
                               zeros)
        return 0

    def bump(u, b, matched=None):
        plsc.addupdate_scatter(hist_v, [b, splat(u * LANES) + lane], ones,
                               mask=matched)
        plsc.addupdate_scatter(coarse_v, [b >> 4, splat(u * LANES) + lane],
                               ones, mask=matched)

    def slot_sum(ref, idx):
        c = plsc.load_gather(ref, [idx, lane])
        for u in range(1, _UNROLL):
            c = c + plsc.load_gather(ref, [idx, splat(u * LANES) + lane])
        return c

    def scan_hist(kneed):
        def cbody(i, carry):
            cum, selc, cumbef, found = carry
            bb = splat(15 - i)
            c = slot_sum(coarse_v, bb)
            hit = jnp.logical_and(found == 0, (cum + c) >= kneed)
            selc = jnp.where(hit, bb, selc)
            cumbef = jnp.where(hit, cum, cumbef)
            found = jnp.where(hit, ones, found)
            return cum + c, selc, cumbef, found

        _, selc, cumbef_c, _ = lax.fori_loop(
            0, 16, cbody, (zeros, zeros, zeros, zeros))
        kneed_f = kneed - cumbef_c

        def fbody(i, carry):
            cum, self_, cumbef, found = carry
            tt = splat(15 - i)
            c = slot_sum(hist_v, selc * 16 + tt)
            hit = jnp.logical_and(found == 0, (cum + c) >= kneed_f)
            self_ = jnp.where(hit, tt, self_)
            cumbef = jnp.where(hit, cum, cumbef)
            found = jnp.where(hit, ones, found)
            return cum + c, self_, cumbef, found

        _, self_, cumbef_f, _ = lax.fori_loop(
            0, 16, fbody, (zeros, zeros, zeros, zeros))
        return selc * 16 + self_, cumbef_c + cumbef_f

    lax.fori_loop(0, 256, zero_hists, 0)
    lax.fori_loop(0, 16, zero_coarse, 0)

    def l0(i, _):
        for u in range(_UNROLL):
            j = i * _UNROLL + u
            x = plsc.load_gather(t_v, [lane, splat(j)]) + 0.0
            b = plsc.bitcast(x, jnp.int32)
            key = jnp.where(b >= 0, b, b ^ jnp.int32(0x7FFFFFFF))
            plsc.store_scatter(k_v, [lane, splat(j)], key)
            bump(u, (key >> 24) + 128)
        return 0

    lax.fori_loop(0, NODE_NUM // _UNROLL, l0, 0)
    sel1, cumbef = scan_hist(kvec)
    total_gt = cumbef

    def refine(shift, want):
        lax.fori_loop(0, 256, zero_hists, 0)
        lax.fori_loop(0, 16, zero_coarse, 0)

        def body(i, _):
            for u in range(_UNROLL):
                j = i * _UNROLL + u
                key = plsc.load_gather(k_v, [lane, splat(j)])
                matched = (key >> (shift + 8)) == want
                bump(u, (key >> shift) & 0xFF, matched)
            return 0

        lax.fori_loop(0, NODE_NUM // _UNROLL, body, 0)
        return scan_hist(kvec - total_gt)

    want1 = sel1 - 128
    sel2, cumbef = refine(16, want1)
    total_gt = total_gt + cumbef
    want2 = (want1 << 8) + sel2
    sel3, cumbef = refine(8, want2)
    total_gt = total_gt + cumbef
    want3 = (want2 << 8) + sel3
    sel4, cumbef = refine(0, want3)
    total_gt = total_gt + cumbef

    thr = (want3 << 8) + sel4
    need = kvec - total_gt

    def final(i, run):
        for u in range(_UNROLL):
            j = i * _UNROLL + u
            key = plsc.load_gather(k_v, [lane, splat(j)])
            gt = key > thr
            eq = key == thr
            take = jnp.logical_or(gt, jnp.logical_and(eq, run < need))
            take = jnp.logical_or(take, splat(j) == base + lane)
            plsc.store_scatter(out_v, [lane, splat(j)],
                               jnp.where(take, 1.0, 0.0).astype(jnp.float32))
            run = run + eq.astype(jnp.int32)
        return run

    lax.fori_loop(0, NODE_NUM // _UNROLL, final, zeros)
    pltpu.sync_copy(out_v, mask_hbm.at[pl.ds(base, ROWS_PER_W), :])


_sc_topk = functools.partial(
    pl.kernel,
    out_type=jax.ShapeDtypeStruct((NODE_NUM, NODE_NUM), jnp.float32),
    mesh=plsc.VectorSubcoreMesh(core_axis_name="c", subcore_axis_name="s",
                                num_cores=SC_CORES, num_subcores=SC_SUBCORES),
    scratch_types=[
        pltpu.VMEM((ROWS_PER_W, NODE_NUM), jnp.float32),
        pltpu.VMEM((ROWS_PER_W, NODE_NUM), jnp.int32),
        pltpu.VMEM((256, LANES * _UNROLL), jnp.int32),
        pltpu.VMEM((16, LANES * _UNROLL), jnp.int32),
        pltpu.VMEM((ROWS_PER_W, NODE_NUM), jnp.float32),
    ],
    compiler_params=pltpu.CompilerParams(needs_layout_passes=False),
)(_sc_topk_body)


def _fwd_kernel(data_ref, mask_ref, emb_ref, lin_wT_ref, att_i_ref, att_j_ref,
                att_em_i_ref, att_em_j_ref, gl_bias_ref, bn1_g_ref, bn1_b_ref,
                bno_g_ref, bno_b_ref, w1T_ref, b1_ref, bn2_g_ref, bn2_b_ref,
                w2_ref, b2_ref, out_ref):
    d = data_ref[0]
    emb = emb_ref[...]
    xl = jnp.dot(d, lin_wT_ref[...], preferred_element_type=jnp.float32)

    a = (jnp.sum(xl * att_i_ref[...], axis=1, keepdims=True)
         + jnp.sum(emb * att_em_i_ref[...], axis=1, keepdims=True))
    b = (jnp.sum(xl * att_j_ref[...], axis=1, keepdims=True)
         + jnp.sum(emb * att_em_j_ref[...], axis=1, keepdims=True))

    alpha = a + b.T
    alpha = jnp.where(alpha >= 0, alpha, 0.2 * alpha)
    valid = mask_ref[...] > 0.0
    am = jnp.where(valid, alpha, NEG_INF)
    amax = jnp.max(am, axis=1, keepdims=True)
    p = jnp.exp(am - amax)
    att = p / (jnp.sum(p, axis=1, keepdims=True) + 1e-16)

    agg = jnp.dot(att, xl, preferred_element_type=jnp.float32)
    out = agg + gl_bias_ref[...]
    out = out * (bn1_g_ref[...] * lax.rsqrt(1.0 + EPS)) + bn1_b_ref[...]
    out = jnp.maximum(out, 0.0)

    xo = out * emb
    xo = xo * (bno_g_ref[...] * lax.rsqrt(1.0 + EPS)) + bno_b_ref[...]
    xo = jnp.maximum(xo, 0.0)

    h = jnp.dot(xo, w1T_ref[...], preferred_element_type=jnp.float32) + b1_ref[...]
    h = h * (bn2_g_ref[...] * lax.rsqrt(1.0 + EPS)) + bn2_b_ref[...]
    h = jnp.maximum(h, 0.0)

    y = lax.dot_general(w2_ref[...], h, (((1,), (1,)), ((), ())),
                        preferred_element_type=jnp.float32)
    out_ref[0] = y + b2_ref[...]


def kernel(data, emb, lin_w, att_i, att_j, att_em_i, att_em_j, gl_bias,
           bn1_g, bn1_b, bno_g, bno_b, w1, b1, bn2_g, bn2_b, w2, b2):
    cos = pl.pallas_call(
        _cos_kernel,
        out_shape=jax.ShapeDtypeStruct((NODE_NUM, NODE_NUM), jnp.float32),
    )(emb)
    mask = _sc_topk(cos)

    data3 = data.reshape(BATCH, NODE_NUM, INPUT_DIM)
    data3 = jnp.pad(data3, ((0, 0), (0, 0), (0, 8 - INPUT_DIM)))
    lin_wT = jnp.pad(lin_w.T, ((0, 8 - INPUT_DIM), (0, 0)))

    row = lambda v: v.reshape(1, -1)
    grid_spec = pl.GridSpec(
        grid=(BATCH,),
        in_specs=[
            pl.BlockSpec((1, NODE_NUM, 8), lambda b: (b, 0, 0)),
            pl.BlockSpec((NODE_NUM, NODE_NUM), lambda b: (0, 0)),
            pl.BlockSpec((NODE_NUM, DIM), lambda b: (0, 0)),
            pl.BlockSpec((8, DIM), lambda b: (0, 0)),
        ] + [pl.BlockSpec((1, DIM), lambda b: (0, 0))] * 9 + [
            pl.BlockSpec((DIM, INTER), lambda b: (0, 0)),
            pl.BlockSpec((1, INTER), lambda b: (0, 0)),
            pl.BlockSpec((1, INTER), lambda b: (0, 0)),
            pl.BlockSpec((1, INTER), lambda b: (0, 0)),
            pl.BlockSpec((1, INTER), lambda b: (0, 0)),
            pl.BlockSpec((1, 1), lambda b: (0, 0)),
        ],
        out_specs=pl.BlockSpec((1, 1, NODE_NUM), lambda b: (b, 0, 0)),
    )
    out = pl.pallas_call(
        _fwd_kernel,
        grid_spec=grid_spec,
        out_shape=jax.ShapeDtypeStruct((BATCH, 1, NODE_NUM), jnp.float32),
        compiler_params=pltpu.CompilerParams(
            dimension_semantics=("arbitrary",),
        ),
    )(data3, mask, emb, lin_wT, row(att_i), row(att_j), row(att_em_i),
      row(att_em_j), row(gl_bias), row(bn1_g), row(bn1_b), row(bno_g),
      row(bno_b), w1.T, row(b1), row(bn2_g), row(bn2_b), w2, b2.reshape(1, 1))
    return out.reshape(BATCH, NODE_NUM)


# speedup vs baseline: 1.1857x; 1.1857x over previous
import functools

import jax
import jax.numpy as jnp
from jax import lax
from jax.experimental import pallas as pl
from jax.experimental.pallas import tpu as pltpu
from jax.experimental.pallas import tpu_sc as plsc

NODE_NUM = 512
DIM = 128
INPUT_DIM = 5
TOPK = 30
BATCH = 32
INTER = 256
EPS = 1e-5
NEG_INF = float("-inf")

SC_CORES = 2
SC_SUBCORES = 16
NUM_WORKERS = SC_CORES * SC_SUBCORES
ROWS_PER_W = NODE_NUM // NUM_WORKERS
LANES = 16


def _cos_kernel(emb_ref, cos_ref):
    w = emb_ref[...]
    g = jnp.dot(w, w.T, preferred_element_type=jnp.float32)
    nrm = jnp.sqrt(jnp.sum(w * w, axis=1, keepdims=True))
    cos_ref[...] = g / (nrm * nrm.T)


_UNROLL = 8


def _sc_topk_body(cos_hbm, mask_hbm, t_v, k_v, hist_v, out_v):
    wid = lax.axis_index("s") * SC_CORES + lax.axis_index("c")
    base = wid * ROWS_PER_W
    pltpu.sync_copy(cos_hbm.at[pl.ds(base, ROWS_PER_W), :], t_v)

    lane = lax.iota(jnp.int32, LANES)
    ones = jnp.full((LANES,), 1, jnp.int32)
    zeros = jnp.full((LANES,), 0, jnp.int32)
    kvec = jnp.full((LANES,), TOPK, jnp.int32)

    def splat(v):
        return jnp.full((LANES,), v, jnp.int32)

    def zero_hists(i, _):
        for u in range(_UNROLL):
            plsc.store_scatter(hist_v, [splat(i), splat(u * LANES) + lane],
                               zeros)
        return 0

    def bump(u, b, matched=None):
        plsc.addupdate_scatter(hist_v, [b, splat(u * LANES) + lane], ones,
                               mask=matched)

    def scan_hist(kneed):
        def body(i, carry):
            cum, selb, cumbef, found = carry
            bb = splat(255 - i)
            c = plsc.load_gather(hist_v, [bb, lane])
            for u in range(1, _UNROLL):
                c = c + plsc.load_gather(hist_v, [bb, splat(u * LANES) + lane])
            hit = jnp.logical_and(found == 0, (cum + c) >= kneed)
            selb = jnp.where(hit, bb, selb)
            cumbef = jnp.where(hit, cum, cumbef)
            found = jnp.where(hit, ones, found)
            return cum + c, selb, cumbef, found

        _, selb, cumbef, _ = lax.fori_loop(
            0, 256, body, (zeros, zeros, zeros, zeros))
        return selb, cumbef

    lax.fori_loop(0, 256, zero_hists, 0)

    def l0(i, _):
        for u in range(_UNROLL):
            j = i * _UNROLL + u
            x = plsc.load_gather(t_v, [lane, splat(j)]) + 0.0
            b = plsc.bitcast(x, jnp.int32)
            key = jnp.where(b >= 0, b, b ^ jnp.int32(0x7FFFFFFF))
            plsc.store_scatter(k_v, [splat(j), lane], key)
            bump(u, (key >> 24) + 128)
        return 0

    lax.fori_loop(0, NODE_NUM // _UNROLL, l0, 0)
    sel1, cumbef = scan_hist(kvec)
    total_gt = cumbef

    def refine(shift, want):
        lax.fori_loop(0, 256, zero_hists, 0)

        def body(i, _):
            for u in range(_UNROLL):
                j = i * _UNROLL + u
                key = plsc.load_gather(k_v, [splat(j), lane])
                matched = (key >> (shift + 8)) == want
                bump(u, (key >> shift) & 0xFF, matched)
            return 0

        lax.fori_loop(0, NODE_NUM // _UNROLL, body, 0)
        return scan_hist(kvec - total_gt)

    want1 = sel1 - 128
    sel2, cumbef = refine(16, want1)
    total_gt = total_gt + cumbef
    want2 = (want1 << 8) + sel2
    sel3, cumbef = refine(8, want2)
    total_gt = total_gt + cumbef
    want3 = (want2 << 8) + sel3
    sel4, cumbef = refine(0, want3)
    total_gt = total_gt + cumbef

    thr = (want3 << 8) + sel4
    need = kvec - total_gt

    def final(i, run):
        for u in range(_UNROLL):
            j = i * _UNROLL + u
            key = plsc.load_gather(k_v, [splat(j), lane])
            gt = key > thr
            eq = key == thr
            take = jnp.logical_or(gt, jnp.logical_and(eq, run < need))
            take = jnp.logical_or(take, splat(j) == base + lane)
            plsc.store_scatter(out_v, [lane, splat(j)],
                               jnp.where(take, 1.0, 0.0).astype(jnp.float32))
            run = run + eq.astype(jnp.int32)
        return run

    lax.fori_loop(0, NODE_NUM // _UNROLL, final, zeros)
    pltpu.sync_copy(out_v, mask_hbm.at[pl.ds(base, ROWS_PER_W), :])


_sc_topk = functools.partial(
    pl.kernel,
    out_type=jax.ShapeDtypeStruct((NODE_NUM, NODE_NUM), jnp.float32),
    mesh=plsc.VectorSubcoreMesh(core_axis_name="c", subcore_axis_name="s",
                                num_cores=SC_CORES, num_subcores=SC_SUBCORES),
    scratch_types=[
        pltpu.VMEM((ROWS_PER_W, NODE_NUM), jnp.float32),
        pltpu.VMEM((NODE_NUM, LANES), jnp.int32),
        pltpu.VMEM((256, LANES * _UNROLL), jnp.int32),
        pltpu.VMEM((ROWS_PER_W, NODE_NUM), jnp.float32),
    ],
    compiler_params=pltpu.CompilerParams(needs_layout_passes=False),
)(_sc_topk_body)


def _fwd_kernel(data_ref, mask_ref, emb_ref, lin_wT_ref, att_i_ref, att_j_ref,
                att_em_i_ref, att_em_j_ref, gl_bias_ref, bn1_g_ref, bn1_b_ref,
                bno_g_ref, bno_b_ref, w1T_ref, b1_ref, bn2_g_ref, bn2_b_ref,
                w2_ref, b2_ref, out_ref):
    d = data_ref[0]
    emb = emb_ref[...]
    xl = jnp.dot(d, lin_wT_ref[...], preferred_element_type=jnp.float32)

    a = (jnp.sum(xl * att_i_ref[...], axis=1, keepdims=True)
         + jnp.sum(emb * att_em_i_ref[...], axis=1, keepdims=True))
    b = (jnp.sum(xl * att_j_ref[...], axis=1, keepdims=True)
         + jnp.sum(emb * att_em_j_ref[...], axis=1, keepdims=True))

    alpha = a + b.T
    alpha = jnp.where(alpha >= 0, alpha, 0.2 * alpha)
    valid = mask_ref[...] > 0.0
    am = jnp.where(valid, alpha, NEG_INF)
    amax = jnp.max(am, axis=1, keepdims=True)
    p = jnp.exp(am - amax)
    att = p / (jnp.sum(p, axis=1, keepdims=True) + 1e-16)

    agg = jnp.dot(att, xl, preferred_element_type=jnp.float32)
    out = agg + gl_bias_ref[...]
    out = out * (bn1_g_ref[...] * lax.rsqrt(1.0 + EPS)) + bn1_b_ref[...]
    out = jnp.maximum(out, 0.0)

    xo = out * emb
    xo = xo * (bno_g_ref[...] * lax.rsqrt(1.0 + EPS)) + bno_b_ref[...]
    xo = jnp.maximum(xo, 0.0)

    h = jnp.dot(xo, w1T_ref[...], preferred_element_type=jnp.float32) + b1_ref[...]
    h = h * (bn2_g_ref[...] * lax.rsqrt(1.0 + EPS)) + bn2_b_ref[...]
    h = jnp.maximum(h, 0.0)

    y = lax.dot_general(w2_ref[...], h, (((1,), (1,)), ((), ())),
                        preferred_element_type=jnp.float32)
    out_ref[0] = y + b2_ref[...]


def kernel(data, emb, lin_w, att_i, att_j, att_em_i, att_em_j, gl_bias,
           bn1_g, bn1_b, bno_g, bno_b, w1, b1, bn2_g, bn2_b, w2, b2):
    cos = pl.pallas_call(
        _cos_kernel,
        out_shape=jax.ShapeDtypeStruct((NODE_NUM, NODE_NUM), jnp.float32),
    )(emb)
    mask = _sc_topk(cos)

    data3 = data.reshape(BATCH, NODE_NUM, INPUT_DIM)
    data3 = jnp.pad(data3, ((0, 0), (0, 0), (0, 8 - INPUT_DIM)))
    lin_wT = jnp.pad(lin_w.T, ((0, 8 - INPUT_DIM), (0, 0)))

    row = lambda v: v.reshape(1, -1)
    grid_spec = pl.GridSpec(
        grid=(BATCH,),
        in_specs=[
            pl.BlockSpec((1, NODE_NUM, 8), lambda b: (b, 0, 0)),
            pl.BlockSpec((NODE_NUM, NODE_NUM), lambda b: (0, 0)),
            pl.BlockSpec((NODE_NUM, DIM), lambda b: (0, 0)),
            pl.BlockSpec((8, DIM), lambda b: (0, 0)),
        ] + [pl.BlockSpec((1, DIM), lambda b: (0, 0))] * 9 + [
            pl.BlockSpec((DIM, INTER), lambda b: (0, 0)),
            pl.BlockSpec((1, INTER), lambda b: (0, 0)),
            pl.BlockSpec((1, INTER), lambda b: (0, 0)),
            pl.BlockSpec((1, INTER), lambda b: (0, 0)),
            pl.BlockSpec((1, INTER), lambda b: (0, 0)),
            pl.BlockSpec((1, 1), lambda b: (0, 0)),
        ],
        out_specs=pl.BlockSpec((1, 1, NODE_NUM), lambda b: (b, 0, 0)),
    )
    out = pl.pallas_call(
        _fwd_kernel,
        grid_spec=grid_spec,
        out_shape=jax.ShapeDtypeStruct((BATCH, 1, NODE_NUM), jnp.float32),
        compiler_params=pltpu.CompilerParams(
            dimension_semantics=("arbitrary",),
        ),
    )(data3, mask, emb, lin_wT, row(att_i), row(att_j), row(att_em_i),
      row(att_em_j), row(gl_bias), row(bn1_g), row(bn1_b), row(bno_g),
      row(bno_b), w1.T, row(b1), row(bn2_g), row(bn2_b), w2, b2.reshape(1, 1))
    return out.reshape(BATCH, NODE_NUM)


# speedup vs baseline: 1.2560x; 1.0593x over previous
import functools

import jax
import jax.numpy as jnp
from jax import lax
from jax.experimental import pallas as pl
from jax.experimental.pallas import tpu as pltpu
from jax.experimental.pallas import tpu_sc as plsc

NODE_NUM = 512
DIM = 128
INPUT_DIM = 5
TOPK = 30
BATCH = 32
INTER = 256
EPS = 1e-5
NEG_INF = float("-inf")

SC_CORES = 2
SC_SUBCORES = 16
NUM_WORKERS = SC_CORES * SC_SUBCORES
ROWS_PER_W = NODE_NUM // NUM_WORKERS
LANES = 16


def _cos_kernel(emb_ref, cos_ref):
    w = emb_ref[...]
    g = jnp.dot(w, w.T, preferred_element_type=jnp.float32)
    nrm = jnp.sqrt(jnp.sum(w * w, axis=1, keepdims=True))
    cos_ref[...] = g / (nrm * nrm.T)


_UNROLL = 4


def _sc_topk_body(cos_hbm, mask_hbm, t_v, k_v, hist_v, coarse_v, out_v):
    wid = lax.axis_index("s") * SC_CORES + lax.axis_index("c")
    base = wid * ROWS_PER_W
    pltpu.sync_copy(cos_hbm.at[pl.ds(base, ROWS_PER_W), :], t_v)

    lane = lax.iota(jnp.int32, LANES)
    ones = jnp.full((LANES,), 1, jnp.int32)
    zeros = jnp.full((LANES,), 0, jnp.int32)
    kvec = jnp.full((LANES,), TOPK, jnp.int32)

    def splat(v):
        return jnp.full((LANES,), v, jnp.int32)

    def zero_hists(i, _):
        for u in range(8):
            plsc.store_scatter(hist_v, [splat(i * 8 + u), lane], zeros)
        return 0

    def zero_coarse(i, _):
        plsc.store_scatter(coarse_v, [splat(i), lane], zeros)
        return 0

    def bump(b, matched=None):
        plsc.addupdate_scatter(hist_v, [b, lane], ones, mask=matched)
        plsc.addupdate_scatter(coarse_v, [b >> 4, lane], ones, mask=matched)

    def scan_hist(kneed):
        def cbody(i, carry):
            cum, selc, cumbef, found = carry
            bb = splat(15 - i)
            c = plsc.load_gather(coarse_v, [bb, lane])
            hit = jnp.logical_and(found == 0, (cum + c) >= kneed)
            selc = jnp.where(hit, bb, selc)
            cumbef = jnp.where(hit, cum, cumbef)
            found = jnp.where(hit, ones, found)
            return cum + c, selc, cumbef, found

        _, selc, cumbef_c, _ = lax.fori_loop(
            0, 16, cbody, (zeros, zeros, zeros, zeros))
        kneed_f = kneed - cumbef_c

        def fbody(i, carry):
            cum, self_, cumbef, found = carry
            tt = splat(15 - i)
            c = plsc.load_gather(hist_v, [selc * 16 + tt, lane])
            hit = jnp.logical_and(found == 0, (cum + c) >= kneed_f)
            self_ = jnp.where(hit, tt, self_)
            cumbef = jnp.where(hit, cum, cumbef)
            found = jnp.where(hit, ones, found)
            return cum + c, self_, cumbef, found

        _, self_, cumbef_f, _ = lax.fori_loop(
            0, 16, fbody, (zeros, zeros, zeros, zeros))
        return selc * 16 + self_, cumbef_c + cumbef_f

    lax.fori_loop(0, 32, zero_hists, 0)
    lax.fori_loop(0, 16, zero_coarse, 0)

    def l0(i, _):
        for u in range(_UNROLL):
            j = i * _UNROLL + u
            x = plsc.load_gather(t_v, [lane, splat(j)]) + 0.0
            b = plsc.bitcast(x, jnp.int32)
            key = jnp.where(b >= 0, b, b ^ jnp.int32(0x7FFFFFFF))
            plsc.store_scatter(k_v, [splat(j), lane], key)
            bump((key >> 24) + 128)
        return 0

    lax.fori_loop(0, NODE_NUM // _UNROLL, l0, 0)
    sel1, cumbef = scan_hist(kvec)
    total_gt = cumbef

    def refine(shift, want):
        lax.fori_loop(0, 32, zero_hists, 0)
        lax.fori_loop(0, 16, zero_coarse, 0)

        def body(i, _):
            for u in range(_UNROLL):
                j = i * _UNROLL + u
                key = plsc.load_gather(k_v, [splat(j), lane])
                matched = (key >> (shift + 8)) == want
                bump((key >> shift) & 0xFF, matched)
            return 0

        lax.fori_loop(0, NODE_NUM // _UNROLL, body, 0)
        return scan_hist(kvec - total_gt)

    want1 = sel1 - 128
    sel2, cumbef = refine(16, want1)
    total_gt = total_gt + cumbef
    want2 = (want1 << 8) + sel2
    sel3, cumbef = refine(8, want2)
    total_gt = total_gt + cumbef
    want3 = (want2 << 8) + sel3
    sel4, cumbef = refine(0, want3)
    total_gt = total_gt + cumbef

    thr = (want3 << 8) + sel4
    need = kvec - total_gt

    def final(i, run):
        for u in range(_UNROLL):
            j = i * _UNROLL + u
            key = plsc.load_gather(k_v, [splat(j), lane])
            gt = key > thr
            eq = key == thr
            take = jnp.logical_or(gt, jnp.logical_and(eq, run < need))
            take = jnp.logical_or(take, splat(j) == base + lane)
            plsc.store_scatter(out_v, [lane, splat(j)],
                               jnp.where(take, 1.0, 0.0).astype(jnp.float32))
            run = run + eq.astype(jnp.int32)
        return run

    lax.fori_loop(0, NODE_NUM // _UNROLL, final, zeros)
    pltpu.sync_copy(out_v, mask_hbm.at[pl.ds(base, ROWS_PER_W), :])


_sc_topk = functools.partial(
    pl.kernel,
    out_type=jax.ShapeDtypeStruct((NODE_NUM, NODE_NUM), jnp.float32),
    mesh=plsc.VectorSubcoreMesh(core_axis_name="c", subcore_axis_name="s",
                                num_cores=SC_CORES, num_subcores=SC_SUBCORES),
    scratch_types=[
        pltpu.VMEM((ROWS_PER_W, NODE_NUM), jnp.float32),
        pltpu.VMEM((NODE_NUM, LANES), jnp.int32),
        pltpu.VMEM((256, LANES), jnp.int32),
        pltpu.VMEM((16, LANES), jnp.int32),
        pltpu.VMEM((ROWS_PER_W, NODE_NUM), jnp.float32),
    ],
    compiler_params=pltpu.CompilerParams(needs_layout_passes=False),
)(_sc_topk_body)


def _fwd_kernel(data_ref, mask_ref, emb_ref, lin_wT_ref, att_i_ref, att_j_ref,
                att_em_i_ref, att_em_j_ref, gl_bias_ref, bn1_g_ref, bn1_b_ref,
                bno_g_ref, bno_b_ref, w1T_ref, b1_ref, bn2_g_ref, bn2_b_ref,
                w2_ref, b2_ref, out_ref):
    d = data_ref[0]
    emb = emb_ref[...]
    xl = jnp.dot(d, lin_wT_ref[...], preferred_element_type=jnp.float32)

    a = (jnp.sum(xl * att_i_ref[...], axis=1, keepdims=True)
         + jnp.sum(emb * att_em_i_ref[...], axis=1, keepdims=True))
    b = (jnp.sum(xl * att_j_ref[...], axis=1, keepdims=True)
         + jnp.sum(emb * att_em_j_ref[...], axis=1, keepdims=True))

    alpha = a + b.T
    alpha = jnp.where(alpha >= 0, alpha, 0.2 * alpha)
    valid = mask_ref[...] > 0.0
    am = jnp.where(valid, alpha, NEG_INF)
    amax = jnp.max(am, axis=1, keepdims=True)
    p = jnp.exp(am - amax)
    att = p / (jnp.sum(p, axis=1, keepdims=True) + 1e-16)

    agg = jnp.dot(att, xl, preferred_element_type=jnp.float32)
    out = agg + gl_bias_ref[...]
    out = out * (bn1_g_ref[...] * lax.rsqrt(1.0 + EPS)) + bn1_b_ref[...]
    out = jnp.maximum(out, 0.0)

    xo = out * emb
    xo = xo * (bno_g_ref[...] * lax.rsqrt(1.0 + EPS)) + bno_b_ref[...]
    xo = jnp.maximum(xo, 0.0)

    h = jnp.dot(xo, w1T_ref[...], preferred_element_type=jnp.float32) + b1_ref[...]
    h = h * (bn2_g_ref[...] * lax.rsqrt(1.0 + EPS)) + bn2_b_ref[...]
    h = jnp.maximum(h, 0.0)

    y = lax.dot_general(w2_ref[...], h, (((1,), (1,)), ((), ())),
                        preferred_element_type=jnp.float32)
    out_ref[0] = y + b2_ref[...]


def kernel(data, emb, lin_w, att_i, att_j, att_em_i, att_em_j, gl_bias,
           bn1_g, bn1_b, bno_g, bno_b, w1, b1, bn2_g, bn2_b, w2, b2):
    cos = pl.pallas_call(
        _cos_kernel,
        out_shape=jax.ShapeDtypeStruct((NODE_NUM, NODE_NUM), jnp.float32),
    )(emb)
    mask = _sc_topk(cos)

    data3 = data.reshape(BATCH, NODE_NUM, INPUT_DIM)
    data3 = jnp.pad(data3, ((0, 0), (0, 0), (0, 8 - INPUT_DIM)))
    lin_wT = jnp.pad(lin_w.T, ((0, 8 - INPUT_DIM), (0, 0)))

    row = lambda v: v.reshape(1, -1)
    grid_spec = pl.GridSpec(
        grid=(BATCH,),
        in_specs=[
            pl.BlockSpec((1, NODE_NUM, 8), lambda b: (b, 0, 0)),
            pl.BlockSpec((NODE_NUM, NODE_NUM), lambda b: (0, 0)),
            pl.BlockSpec((NODE_NUM, DIM), lambda b: (0, 0)),
            pl.BlockSpec((8, DIM), lambda b: (0, 0)),
        ] + [pl.BlockSpec((1, DIM), lambda b: (0, 0))] * 9 + [
            pl.BlockSpec((DIM, INTER), lambda b: (0, 0)),
            pl.BlockSpec((1, INTER), lambda b: (0, 0)),
            pl.BlockSpec((1, INTER), lambda b: (0, 0)),
            pl.BlockSpec((1, INTER), lambda b: (0, 0)),
            pl.BlockSpec((1, INTER), lambda b: (0, 0)),
            pl.BlockSpec((1, 1), lambda b: (0, 0)),
        ],
        out_specs=pl.BlockSpec((1, 1, NODE_NUM), lambda b: (b, 0, 0)),
    )
    out = pl.pallas_call(
        _fwd_kernel,
        grid_spec=grid_spec,
        out_shape=jax.ShapeDtypeStruct((BATCH, 1, NODE_NUM), jnp.float32),
        compiler_params=pltpu.CompilerParams(
            dimension_semantics=("arbitrary",),
        ),
    )(data3, mask, emb, lin_wT, row(att_i), row(att_j), row(att_em_i),
      row(att_em_j), row(gl_bias), row(bn1_g), row(bn1_b), row(bno_g),
      row(bno_b), w1.T, row(b1), row(bn2_g), row(bn2_b), w2, b2.reshape(1, 1))
    return out.reshape(BATCH, NODE_NUM)


# speedup vs baseline: 1.2752x; 1.0153x over previous
import functools

import jax
import jax.numpy as jnp
from jax import lax
from jax.experimental import pallas as pl
from jax.experimental.pallas import tpu as pltpu
from jax.experimental.pallas import tpu_sc as plsc

NODE_NUM = 512
DIM = 128
INPUT_DIM = 5
TOPK = 30
BATCH = 32
INTER = 256
EPS = 1e-5
NEG_INF = float("-inf")

SC_CORES = 2
SC_SUBCORES = 16
NUM_WORKERS = SC_CORES * SC_SUBCORES
ROWS_PER_W = NODE_NUM // NUM_WORKERS
LANES = 16


def _cos_kernel(emb_ref, cos_ref):
    w = emb_ref[...]
    g = jnp.dot(w, w.T, preferred_element_type=jnp.float32)
    nrm = jnp.sqrt(jnp.sum(w * w, axis=1, keepdims=True))
    cos_ref[...] = g / (nrm * nrm.T)


_UNROLL = 4


def _sc_topk_body(cos_hbm, mask_hbm, t_v, k_v, hist_v, coarse_v, out_v):
    wid = lax.axis_index("s") * SC_CORES + lax.axis_index("c")
    base = wid * ROWS_PER_W
    pltpu.sync_copy(cos_hbm.at[pl.ds(base, ROWS_PER_W), :], t_v)

    lane = lax.iota(jnp.int32, LANES)
    ones = jnp.full((LANES,), 1, jnp.int32)
    zeros = jnp.full((LANES,), 0, jnp.int32)
    kvec = jnp.full((LANES,), TOPK, jnp.int32)

    def splat(v):
        return jnp.full((LANES,), v, jnp.int32)

    def zero_hists(i, _):
        for u in range(8):
            plsc.store_scatter(hist_v, [splat(i * 8 + u), lane], zeros)
        return 0

    def zero_coarse(i, _):
        plsc.store_scatter(coarse_v, [splat(i), lane], zeros)
        return 0

    def bump(b, matched=None):
        plsc.addupdate_scatter(hist_v, [b, lane], ones, mask=matched)
        plsc.addupdate_scatter(coarse_v, [b >> 4, lane], ones, mask=matched)

    def scan_hist(kneed):
        def cbody(i, carry):
            cum, selc, cumbef, found = carry
            bb = splat(15 - i)
            c = plsc.load_gather(coarse_v, [bb, lane])
            hit = jnp.logical_and(found == 0, (cum + c) >= kneed)
            selc = jnp.where(hit, bb, selc)
            cumbef = jnp.where(hit, cum, cumbef)
            found = jnp.where(hit, ones, found)
            return cum + c, selc, cumbef, found

        _, selc, cumbef_c, _ = lax.fori_loop(
            0, 16, cbody, (zeros, zeros, zeros, zeros))
        kneed_f = kneed - cumbef_c

        def fbody(i, carry):
            cum, self_, cumbef, found = carry
            tt = splat(15 - i)
            c = plsc.load_gather(hist_v, [selc * 16 + tt, lane])
            hit = jnp.logical_and(found == 0, (cum + c) >= kneed_f)
            self_ = jnp.where(hit, tt, self_)
            cumbef = jnp.where(hit, cum, cumbef)
            found = jnp.where(hit, ones, found)
            return cum + c, self_, cumbef, found

        _, self_, cumbef_f, _ = lax.fori_loop(
            0, 16, fbody, (zeros, zeros, zeros, zeros))
        return selc * 16 + self_, cumbef_c + cumbef_f

    lax.fori_loop(0, 32, zero_hists, 0)
    lax.fori_loop(0, 16, zero_coarse, 0)

    def l0(i, _):
        for u in range(_UNROLL):
            j = i * _UNROLL + u
            x = plsc.load_gather(t_v, [lane, splat(j)]) + 0.0
            b = plsc.bitcast(x, jnp.int32)
            key = jnp.where(b >= 0, b, b ^ jnp.int32(0x7FFFFFFF))
            plsc.store_scatter(k_v, [splat(j), lane], key)
            bump((key >> 24) + 128)
        return 0

    lax.fori_loop(0, NODE_NUM // _UNROLL, l0, 0)
    sel1, cumbef = scan_hist(kvec)
    total_gt = cumbef

    def refine(shift, want):
        lax.fori_loop(0, 32, zero_hists, 0)
        lax.fori_loop(0, 16, zero_coarse, 0)

        def body(i, _):
            for u in range(_UNROLL):
                j = i * _UNROLL + u
                key = plsc.load_gather(k_v, [splat(j), lane])
                matched = (key >> (shift + 8)) == want
                bump((key >> shift) & 0xFF, matched)
            return 0

        lax.fori_loop(0, NODE_NUM // _UNROLL, body, 0)
        return scan_hist(kvec - total_gt)

    want1 = sel1 - 128
    sel2, cumbef = refine(16, want1)
    total_gt = total_gt + cumbef
    want2 = (want1 << 8) + sel2
    sel3, cumbef = refine(8, want2)
    total_gt = total_gt + cumbef
    want3 = (want2 << 8) + sel3
    sel4, cumbef = refine(0, want3)
    total_gt = total_gt + cumbef

    thr = (want3 << 8) + sel4
    need = kvec - total_gt

    def final(i, run):
        for u in range(_UNROLL):
            j = i * _UNROLL + u
            key = plsc.load_gather(k_v, [splat(j), lane])
            gt = key > thr
            eq = key == thr
            take = jnp.logical_or(gt, jnp.logical_and(eq, run < need))
            take = jnp.logical_or(take, splat(j) == base + lane)
            plsc.store_scatter(out_v, [lane, splat(j)],
                               jnp.where(take, 1.0, 0.0).astype(jnp.float32))
            run = run + eq.astype(jnp.int32)
        return run

    lax.fori_loop(0, NODE_NUM // _UNROLL, final, zeros)
    pltpu.sync_copy(out_v, mask_hbm.at[pl.ds(base, ROWS_PER_W), :])


_sc_topk = functools.partial(
    pl.kernel,
    out_type=jax.ShapeDtypeStruct((NODE_NUM, NODE_NUM), jnp.float32),
    mesh=plsc.VectorSubcoreMesh(core_axis_name="c", subcore_axis_name="s",
                                num_cores=SC_CORES, num_subcores=SC_SUBCORES),
    scratch_types=[
        pltpu.VMEM((ROWS_PER_W, NODE_NUM), jnp.float32),
        pltpu.VMEM((NODE_NUM, LANES), jnp.int32),
        pltpu.VMEM((256, LANES), jnp.int32),
        pltpu.VMEM((16, LANES), jnp.int32),
        pltpu.VMEM((ROWS_PER_W, NODE_NUM), jnp.float32),
    ],
    compiler_params=pltpu.CompilerParams(needs_layout_passes=False),
)(_sc_topk_body)


FWD_BB = 2


def _fwd_kernel(data_ref, mask_ref, emb_ref, lin_wT_ref, att_i_ref, att_j_ref,
                att_em_i_ref, att_em_j_ref, gl_bias_ref, bn1_g_ref, bn1_b_ref,
                bno_g_ref, bno_b_ref, w1T_ref, b1_ref, bn2_g_ref, bn2_b_ref,
                w2_ref, b2_ref, out_ref):
    emb = emb_ref[...]
    valid = mask_ref[...] > 0.0
    e_i = jnp.sum(emb * att_em_i_ref[...], axis=1, keepdims=True)
    e_j = jnp.sum(emb * att_em_j_ref[...], axis=1, keepdims=True)
    for u in range(FWD_BB):
        d = data_ref[u]
        xl = jnp.dot(d, lin_wT_ref[...], preferred_element_type=jnp.float32)

        a = jnp.sum(xl * att_i_ref[...], axis=1, keepdims=True) + e_i
        b = jnp.sum(xl * att_j_ref[...], axis=1, keepdims=True) + e_j

        alpha = a + b.T
        alpha = jnp.where(alpha >= 0, alpha, 0.2 * alpha)
        am = jnp.where(valid, alpha, NEG_INF)
        amax = jnp.max(am, axis=1, keepdims=True)
        p = jnp.exp(am - amax)
        att = p / (jnp.sum(p, axis=1, keepdims=True) + 1e-16)

        agg = jnp.dot(att, xl, preferred_element_type=jnp.float32)
        out = agg + gl_bias_ref[...]
        out = out * (bn1_g_ref[...] * lax.rsqrt(1.0 + EPS)) + bn1_b_ref[...]
        out = jnp.maximum(out, 0.0)

        xo = out * emb
        xo = xo * (bno_g_ref[...] * lax.rsqrt(1.0 + EPS)) + bno_b_ref[...]
        xo = jnp.maximum(xo, 0.0)

        h = (jnp.dot(xo, w1T_ref[...], preferred_element_type=jnp.float32)
             + b1_ref[...])
        h = h * (bn2_g_ref[...] * lax.rsqrt(1.0 + EPS)) + bn2_b_ref[...]
        h = jnp.maximum(h, 0.0)

        y = lax.dot_general(w2_ref[...], h, (((1,), (1,)), ((), ())),
                            preferred_element_type=jnp.float32)
        out_ref[u] = y + b2_ref[...]


def kernel(data, emb, lin_w, att_i, att_j, att_em_i, att_em_j, gl_bias,
           bn1_g, bn1_b, bno_g, bno_b, w1, b1, bn2_g, bn2_b, w2, b2):
    cos = pl.pallas_call(
        _cos_kernel,
        out_shape=jax.ShapeDtypeStruct((NODE_NUM, NODE_NUM), jnp.float32),
    )(emb)
    mask = _sc_topk(cos)

    data3 = data.reshape(BATCH, NODE_NUM, INPUT_DIM)
    data3 = jnp.pad(data3, ((0, 0), (0, 0), (0, 8 - INPUT_DIM)))
    lin_wT = jnp.pad(lin_w.T, ((0, 8 - INPUT_DIM), (0, 0)))

    row = lambda v: v.reshape(1, -1)
    grid_spec = pl.GridSpec(
        grid=(BATCH // FWD_BB,),
        in_specs=[
            pl.BlockSpec((FWD_BB, NODE_NUM, 8), lambda b: (b, 0, 0)),
            pl.BlockSpec((NODE_NUM, NODE_NUM), lambda b: (0, 0)),
            pl.BlockSpec((NODE_NUM, DIM), lambda b: (0, 0)),
            pl.BlockSpec((8, DIM), lambda b: (0, 0)),
        ] + [pl.BlockSpec((1, DIM), lambda b: (0, 0))] * 9 + [
            pl.BlockSpec((DIM, INTER), lambda b: (0, 0)),
            pl.BlockSpec((1, INTER), lambda b: (0, 0)),
            pl.BlockSpec((1, INTER), lambda b: (0, 0)),
            pl.BlockSpec((1, INTER), lambda b: (0, 0)),
            pl.BlockSpec((1, INTER), lambda b: (0, 0)),
            pl.BlockSpec((1, 1), lambda b: (0, 0)),
        ],
        out_specs=pl.BlockSpec((FWD_BB, 1, NODE_NUM), lambda b: (b, 0, 0)),
    )
    out = pl.pallas_call(
        _fwd_kernel,
        grid_spec=grid_spec,
        out_shape=jax.ShapeDtypeStruct((BATCH, 1, NODE_NUM), jnp.float32),
        compiler_params=pltpu.CompilerParams(
            dimension_semantics=("arbitrary",),
        ),
    )(data3, mask, emb, lin_wT, row(att_i), row(att_j), row(att_em_i),
      row(att_em_j), row(gl_bias), row(bn1_g), row(bn1_b), row(bno_g),
      row(bno_b), w1.T, row(b1), row(bn2_g), row(bn2_b), w2, b2.reshape(1, 1))
    return out.reshape(BATCH, NODE_NUM)


# speedup vs baseline: 1.2928x; 1.0138x over previous
import functools

import jax
import jax.numpy as jnp
from jax import lax
from jax.experimental import pallas as pl
from jax.experimental.pallas import tpu as pltpu
from jax.experimental.pallas import tpu_sc as plsc

NODE_NUM = 512
DIM = 128
INPUT_DIM = 5
TOPK = 30
BATCH = 32
INTER = 256
EPS = 1e-5
NEG_INF = float("-inf")

SC_CORES = 2
SC_SUBCORES = 16
NUM_WORKERS = SC_CORES * SC_SUBCORES
ROWS_PER_W = NODE_NUM // NUM_WORKERS
LANES = 16


def _cos_kernel(emb_ref, cos_ref):
    w = emb_ref[...]
    g = jnp.dot(w, w.T, preferred_element_type=jnp.float32)
    nrm = jnp.sqrt(jnp.sum(w * w, axis=1, keepdims=True))
    cos_ref[...] = g / (nrm * nrm.T)


_UNROLL = 4


def _sc_topk_body(cos_hbm, mask_hbm, t_v, k_v, hist_v, coarse_v, out_v):
    wid = lax.axis_index("s") * SC_CORES + lax.axis_index("c")
    base = wid * ROWS_PER_W
    pltpu.sync_copy(cos_hbm.at[pl.ds(base, ROWS_PER_W), :], t_v)

    lane = lax.iota(jnp.int32, LANES)
    ones = jnp.full((LANES,), 1, jnp.int32)
    zeros = jnp.full((LANES,), 0, jnp.int32)
    kvec = jnp.full((LANES,), TOPK, jnp.int32)

    def splat(v):
        return jnp.full((LANES,), v, jnp.int32)

    def zero_hists(i, _):
        for u in range(8):
            plsc.store_scatter(hist_v, [splat(i * 8 + u), lane], zeros)
        return 0

    def zero_coarse(i, _):
        plsc.store_scatter(coarse_v, [splat(i), lane], zeros)
        return 0

    def bump(b, matched=None):
        plsc.addupdate_scatter(hist_v, [b, lane], ones, mask=matched)
        plsc.addupdate_scatter(coarse_v, [b >> 4, lane], ones, mask=matched)

    def scan_hist(kneed):
        def cbody(i, carry):
            cum, selc, cumbef, found = carry
            bb = splat(15 - i)
            c = plsc.load_gather(coarse_v, [bb, lane])
            hit = jnp.logical_and(found == 0, (cum + c) >= kneed)
            selc = jnp.where(hit, bb, selc)
            cumbef = jnp.where(hit, cum, cumbef)
            found = jnp.where(hit, ones, found)
            return cum + c, selc, cumbef, found

        _, selc, cumbef_c, _ = lax.fori_loop(
            0, 16, cbody, (zeros, zeros, zeros, zeros))
        kneed_f = kneed - cumbef_c

        def fbody(i, carry):
            cum, self_, cumbef, found = carry
            tt = splat(15 - i)
            c = plsc.load_gather(hist_v, [selc * 16 + tt, lane])
            hit = jnp.logical_and(found == 0, (cum + c) >= kneed_f)
            self_ = jnp.where(hit, tt, self_)
            cumbef = jnp.where(hit, cum, cumbef)
            found = jnp.where(hit, ones, found)
            return cum + c, self_, cumbef, found

        _, self_, cumbef_f, _ = lax.fori_loop(
            0, 16, fbody, (zeros, zeros, zeros, zeros))
        return selc * 16 + self_, cumbef_c + cumbef_f

    lax.fori_loop(0, 32, zero_hists, 0)
    lax.fori_loop(0, 16, zero_coarse, 0)

    def l0(i, _):
        for u in range(_UNROLL):
            j = i * _UNROLL + u
            x = plsc.load_gather(t_v, [lane, splat(j)]) + 0.0
            b = plsc.bitcast(x, jnp.int32)
            key = jnp.where(b >= 0, b, b ^ jnp.int32(0x7FFFFFFF))
            plsc.store_scatter(k_v, [splat(j), lane], key)
            bump((key >> 24) + 128)
        return 0

    lax.fori_loop(0, NODE_NUM // _UNROLL, l0, 0)
    sel1, cumbef = scan_hist(kvec)
    total_gt = cumbef

    def refine(shift, want):
        lax.fori_loop(0, 32, zero_hists, 0)
        lax.fori_loop(0, 16, zero_coarse, 0)

        def body(i, _):
            for u in range(_UNROLL):
                j = i * _UNROLL + u
                key = plsc.load_gather(k_v, [splat(j), lane])
                matched = (key >> (shift + 8)) == want
                bump((key >> shift) & 0xFF, matched)
            return 0

        lax.fori_loop(0, NODE_NUM // _UNROLL, body, 0)
        return scan_hist(kvec - total_gt)

    want1 = sel1 - 128
    sel2, cumbef = refine(16, want1)
    total_gt = total_gt + cumbef
    want2 = (want1 << 8) + sel2
    sel3, cumbef = refine(8, want2)
    total_gt = total_gt + cumbef
    want3 = (want2 << 8) + sel3
    sel4, cumbef = refine(0, want3)
    total_gt = total_gt + cumbef

    thr = (want3 << 8) + sel4
    need = kvec - total_gt

    def final(i, run):
        for u in range(_UNROLL):
            j = i * _UNROLL + u
            key = plsc.load_gather(k_v, [splat(j), lane])
            gt = key > thr
            eq = key == thr
            take = jnp.logical_or(gt, jnp.logical_and(eq, run < need))
            take = jnp.logical_or(take, splat(j) == base + lane)
            plsc.store_scatter(out_v, [lane, splat(j)],
                               jnp.where(take, 1.0, 0.0).astype(jnp.float32))
            run = run + eq.astype(jnp.int32)
        return run

    lax.fori_loop(0, NODE_NUM // _UNROLL, final, zeros)
    pltpu.sync_copy(out_v, mask_hbm.at[pl.ds(base, ROWS_PER_W), :])


_sc_topk = functools.partial(
    pl.kernel,
    out_type=jax.ShapeDtypeStruct((NODE_NUM, NODE_NUM), jnp.float32),
    mesh=plsc.VectorSubcoreMesh(core_axis_name="c", subcore_axis_name="s",
                                num_cores=SC_CORES, num_subcores=SC_SUBCORES),
    scratch_types=[
        pltpu.VMEM((ROWS_PER_W, NODE_NUM), jnp.float32),
        pltpu.VMEM((NODE_NUM, LANES), jnp.int32),
        pltpu.VMEM((256, LANES), jnp.int32),
        pltpu.VMEM((16, LANES), jnp.int32),
        pltpu.VMEM((ROWS_PER_W, NODE_NUM), jnp.float32),
    ],
    compiler_params=pltpu.CompilerParams(needs_layout_passes=False),
)(_sc_topk_body)


FWD_BB = 4


def _fwd_kernel(data_ref, mask_ref, emb_ref, lin_wT_ref, att_i_ref, att_j_ref,
                att_em_i_ref, att_em_j_ref, gl_bias_ref, bn1_g_ref, bn1_b_ref,
                bno_g_ref, bno_b_ref, w1T_ref, b1_ref, bn2_g_ref, bn2_b_ref,
                w2_ref, b2_ref, out_ref):
    emb = emb_ref[...]
    valid = mask_ref[...] > 0.0
    e_i = jnp.sum(emb * att_em_i_ref[...], axis=1, keepdims=True)
    e_j = jnp.sum(emb * att_em_j_ref[...], axis=1, keepdims=True)
    for u in range(FWD_BB):
        d = data_ref[u]
        xl = jnp.dot(d, lin_wT_ref[...], preferred_element_type=jnp.float32)

        a = jnp.sum(xl * att_i_ref[...], axis=1, keepdims=True) + e_i
        b = jnp.sum(xl * att_j_ref[...], axis=1, keepdims=True) + e_j

        alpha = a + b.T
        alpha = jnp.where(alpha >= 0, alpha, 0.2 * alpha)
        am = jnp.where(valid, alpha, NEG_INF)
        amax = jnp.max(am, axis=1, keepdims=True)
        p = jnp.exp(am - amax)
        att = p / (jnp.sum(p, axis=1, keepdims=True) + 1e-16)

        agg = jnp.dot(att, xl, preferred_element_type=jnp.float32)
        out = agg + gl_bias_ref[...]
        out = out * (bn1_g_ref[...] * lax.rsqrt(1.0 + EPS)) + bn1_b_ref[...]
        out = jnp.maximum(out, 0.0)

        xo = out * emb
        xo = xo * (bno_g_ref[...] * lax.rsqrt(1.0 + EPS)) + bno_b_ref[...]
        xo = jnp.maximum(xo, 0.0)

        h = (jnp.dot(xo, w1T_ref[...], preferred_element_type=jnp.float32)
             + b1_ref[...])
        h = h * (bn2_g_ref[...] * lax.rsqrt(1.0 + EPS)) + bn2_b_ref[...]
        h = jnp.maximum(h, 0.0)

        y = lax.dot_general(w2_ref[...], h, (((1,), (1,)), ((), ())),
                            preferred_element_type=jnp.float32)
        out_ref[u] = y + b2_ref[...]


def kernel(data, emb, lin_w, att_i, att_j, att_em_i, att_em_j, gl_bias,
           bn1_g, bn1_b, bno_g, bno_b, w1, b1, bn2_g, bn2_b, w2, b2):
    cos = pl.pallas_call(
        _cos_kernel,
        out_shape=jax.ShapeDtypeStruct((NODE_NUM, NODE_NUM), jnp.float32),
    )(emb)
    mask = _sc_topk(cos)

    data3 = data.reshape(BATCH, NODE_NUM, INPUT_DIM)
    data3 = jnp.pad(data3, ((0, 0), (0, 0), (0, 8 - INPUT_DIM)))
    lin_wT = jnp.pad(lin_w.T, ((0, 8 - INPUT_DIM), (0, 0)))

    row = lambda v: v.reshape(1, -1)
    grid_spec = pl.GridSpec(
        grid=(BATCH // FWD_BB,),
        in_specs=[
            pl.BlockSpec((FWD_BB, NODE_NUM, 8), lambda b: (b, 0, 0)),
            pl.BlockSpec((NODE_NUM, NODE_NUM), lambda b: (0, 0)),
            pl.BlockSpec((NODE_NUM, DIM), lambda b: (0, 0)),
            pl.BlockSpec((8, DIM), lambda b: (0, 0)),
        ] + [pl.BlockSpec((1, DIM), lambda b: (0, 0))] * 9 + [
            pl.BlockSpec((DIM, INTER), lambda b: (0, 0)),
            pl.BlockSpec((1, INTER), lambda b: (0, 0)),
            pl.BlockSpec((1, INTER), lambda b: (0, 0)),
            pl.BlockSpec((1, INTER), lambda b: (0, 0)),
            pl.BlockSpec((1, INTER), lambda b: (0, 0)),
            pl.BlockSpec((1, 1), lambda b: (0, 0)),
        ],
        out_specs=pl.BlockSpec((FWD_BB, 1, NODE_NUM), lambda b: (b, 0, 0)),
    )
    out = pl.pallas_call(
        _fwd_kernel,
        grid_spec=grid_spec,
        out_shape=jax.ShapeDtypeStruct((BATCH, 1, NODE_NUM), jnp.float32),
        compiler_params=pltpu.CompilerParams(
            dimension_semantics=("arbitrary",),
        ),
    )(data3, mask, emb, lin_wT, row(att_i), row(att_j), row(att_em_i),
      row(att_em_j), row(gl_bias), row(bn1_g), row(bn1_b), row(bno_g),
      row(bno_b), w1.T, row(b1), row(bn2_g), row(bn2_b), w2, b2.reshape(1, 1))
    return out.reshape(BATCH, NODE_NUM)
